# baseline (speedup 1.0000x reference)
import jax
import jax.numpy as jnp
from jax import lax
from jax.experimental import pallas as pl
from jax.experimental.pallas import tpu as pltpu
from jax.experimental.pallas import tpu_sc as plsc

NC = 2
NS = 16
NW = NC * NS
L = 16

B_ROWS = 16384
N_FEAT = 26
EMB = 64
B = B_ROWS * N_FEAT
XR_PER_W = B_ROWS // NW
XR_CHUNK = 16
CHUNK = XR_CHUNK * N_FEAT
N_CHUNKS = XR_PER_W // XR_CHUNK
N_GROUPS = CHUNK // L
JW = 4


def _rsqrt_nr(s):
    i = lax.bitcast_convert_type(s, jnp.int32)
    i = jnp.int32(0x5F3759DF) - (i >> 1)
    y = lax.bitcast_convert_type(i, jnp.float32)
    for _ in range(3):
        y = y * (1.5 - 0.5 * s * y * y)
    return y


def _body(
    x_hbm, table_hbm, bias_hbm, out_hbm,
    idx0, idx1, in0, in1, out0, out1, bias_v, bbc,
    sg0, sg1, sw0, sw1,
):
    wid = lax.axis_index("s") * NC + lax.axis_index("c")
    xbase = wid * XR_PER_W
    lanes = lax.iota(jnp.int32, L)

    pltpu.sync_copy(bias_hbm, bias_v)
    for t in range(EMB // L):
        bt = bias_v[pl.ds(t * L, L)]
        for e in range(L):
            bbc[pl.ds((t * L + e) * L, L)] = jnp.full((L,), bt[e], jnp.float32)

    def compute(in_b, out_b):
        @plsc.parallel_loop(0, N_GROUPS, unroll=1)
        def group(g):
            rows = g * L + lanes
            xr = rows // N_FEAT
            ft = rows % N_FEAT
            accs = [jnp.zeros((L,), jnp.float32) for _ in range(JW)]
            for jt in range(0, EMB, JW):
                cols = [jnp.full((L,), jt + k, jnp.int32) for k in range(JW)]
                cs = [
                    plsc.load_gather(in_b, [rows, cols[k]])
                    for k in range(JW)
                ]
                bs = [bbc[pl.ds((jt + k) * L, L)] for k in range(JW)]
                ts = [cs[k] + bs[k] for k in range(JW)]
                accs = [accs[k] + ts[k] * ts[k] for k in range(JW)]
            acc = (accs[0] + accs[1]) + (accs[2] + accs[3])
            rs = _rsqrt_nr(acc)
            inv = 1.0 / jnp.maximum(acc * rs, 1e-12)
            for jt in range(0, EMB, JW):
                cols = [jnp.full((L,), jt + k, jnp.int32) for k in range(JW)]
                cs = [
                    plsc.load_gather(in_b, [rows, cols[k]])
                    for k in range(JW)
                ]
                bs = [bbc[pl.ds((jt + k) * L, L)] for k in range(JW)]
                ts = [(cs[k] + bs[k]) * inv for k in range(JW)]
                for k in range(JW):
                    plsc.store_scatter(out_b, [xr, ft, cols[k]], ts[k])

    bufs = ((idx0, in0, out0, sg0, sw0), (idx1, in1, out1, sg1, sw1))
    base = wid * XR_PER_W * N_FEAT

    pltpu.sync_copy(x_hbm.at[pl.ds(base, CHUNK)], idx0)
    pltpu.make_async_copy(table_hbm.at[idx0], in0, sg0).start()

    def pair_body(cc, carry):
        for b, (idx_b, in_b, out_b, sg_b, sw_b) in enumerate(bufs):
            idx_n, in_n, _, sg_n, _ = bufs[1 - b]
            c = cc * 2 + b
            xoff = xbase + c * XR_CHUNK
            off = base + c * CHUNK

            @pl.when(c + 1 < N_CHUNKS)
            def _():
                pltpu.sync_copy(x_hbm.at[pl.ds(off + CHUNK, CHUNK)], idx_n)
                pltpu.make_async_copy(table_hbm.at[idx_n], in_n, sg_n).start()

            pltpu.make_async_copy(table_hbm.at[idx_b], in_b, sg_b).wait()

            @pl.when(c >= 2)
            def _():
                pltpu.make_async_copy(
                    out_b, out_hbm.at[pl.ds(xoff, XR_CHUNK)], sw_b
                ).wait()

            compute(in_b, out_b)
            pltpu.make_async_copy(
                out_b, out_hbm.at[pl.ds(xoff, XR_CHUNK)], sw_b
            ).start()
        return carry

    lax.fori_loop(0, N_CHUNKS // 2, pair_body, 0)

    for b, (_, _, out_b, _, sw_b) in enumerate(bufs):
        xoff = xbase + (N_CHUNKS - 2 + b) * XR_CHUNK
        pltpu.make_async_copy(
            out_b, out_hbm.at[pl.ds(xoff, XR_CHUNK)], sw_b
        ).wait()


@jax.jit
def kernel(x, table, bias):
    mesh = plsc.VectorSubcoreMesh(core_axis_name="c", subcore_axis_name="s")
    run = pl.kernel(
        _body,
        out_type=jax.ShapeDtypeStruct((B_ROWS, N_FEAT, EMB), jnp.float32),
        mesh=mesh,
        compiler_params=pltpu.CompilerParams(
            needs_layout_passes=False, use_tc_tiling_on_sc=False
        ),
        scratch_types=[
            pltpu.VMEM((CHUNK,), jnp.int32),
            pltpu.VMEM((CHUNK,), jnp.int32),
            pltpu.VMEM((CHUNK, EMB), jnp.float32),
            pltpu.VMEM((CHUNK, EMB), jnp.float32),
            pltpu.VMEM((XR_CHUNK, N_FEAT, EMB), jnp.float32),
            pltpu.VMEM((XR_CHUNK, N_FEAT, EMB), jnp.float32),
            pltpu.VMEM((EMB,), jnp.float32),
            pltpu.VMEM((EMB * L,), jnp.float32),
            pltpu.SemaphoreType.DMA,
            pltpu.SemaphoreType.DMA,
            pltpu.SemaphoreType.DMA,
            pltpu.SemaphoreType.DMA,
        ],
    )
    xf = x.reshape(-1).astype(jnp.int32)
    return run(xf, table, bias)


# speedup vs baseline: 1.0907x; 1.0907x over previous
import jax
import jax.numpy as jnp
from jax import lax
from jax.experimental import pallas as pl
from jax.experimental.pallas import tpu as pltpu
from jax.experimental.pallas import tpu_sc as plsc

NC = 2
NS = 16
NW = NC * NS
L = 16

B_ROWS = 16384
N_FEAT = 26
EMB = 64
B = B_ROWS * N_FEAT
XR_PER_W = B_ROWS // NW
XR_CHUNK = 16
CHUNK = XR_CHUNK * N_FEAT
N_CHUNKS = XR_PER_W // XR_CHUNK
N_GROUPS = CHUNK // L
JW = 8


def _rsqrt_nr(s):
    i = lax.bitcast_convert_type(s, jnp.int32)
    i = jnp.int32(0x5F3759DF) - (i >> 1)
    y = lax.bitcast_convert_type(i, jnp.float32)
    for _ in range(3):
        y = y * (1.5 - 0.5 * s * y * y)
    return y


def _body(
    x_hbm, table_hbm, bias_hbm, out_hbm,
    idx0, idx1, in0, in1, out0, out1, bias_v, bbc,
    sg0, sg1, sw0, sw1,
):
    wid = lax.axis_index("s") * NC + lax.axis_index("c")
    xbase = wid * XR_PER_W
    lanes = lax.iota(jnp.int32, L)

    pltpu.sync_copy(bias_hbm, bias_v)
    for t in range(EMB // L):
        bt = bias_v[pl.ds(t * L, L)]
        for e in range(L):
            bbc[pl.ds((t * L + e) * L, L)] = jnp.full((L,), bt[e], jnp.float32)

    def compute(in_b, out_b):
        @plsc.parallel_loop(0, N_GROUPS, unroll=2)
        def group(g):
            rows = g * L + lanes
            xr = rows // N_FEAT
            ft = rows % N_FEAT
            accs = [jnp.zeros((L,), jnp.float32) for _ in range(JW)]
            for jt in range(0, EMB, JW):
                cols = [jnp.full((L,), jt + k, jnp.int32) for k in range(JW)]
                cs = [
                    plsc.load_gather(in_b, [rows, cols[k]])
                    for k in range(JW)
                ]
                bs = [bbc[pl.ds((jt + k) * L, L)] for k in range(JW)]
                ts = [cs[k] + bs[k] for k in range(JW)]
                accs = [accs[k] + ts[k] * ts[k] for k in range(JW)]
            while len(accs) > 1:
                accs = [accs[i] + accs[i + 1] for i in range(0, len(accs), 2)]
            acc = accs[0]
            rs = _rsqrt_nr(acc)
            inv = 1.0 / jnp.maximum(acc * rs, 1e-12)
            for jt in range(0, EMB, JW):
                cols = [jnp.full((L,), jt + k, jnp.int32) for k in range(JW)]
                cs = [
                    plsc.load_gather(in_b, [rows, cols[k]])
                    for k in range(JW)
                ]
                bs = [bbc[pl.ds((jt + k) * L, L)] for k in range(JW)]
                ts = [(cs[k] + bs[k]) * inv for k in range(JW)]
                for k in range(JW):
                    plsc.store_scatter(out_b, [xr, ft, cols[k]], ts[k])

    bufs = ((idx0, in0, out0, sg0, sw0), (idx1, in1, out1, sg1, sw1))
    base = wid * XR_PER_W * N_FEAT

    pltpu.sync_copy(x_hbm.at[pl.ds(base, CHUNK)], idx0)
    pltpu.make_async_copy(table_hbm.at[idx0], in0, sg0).start()

    def pair_body(cc, carry):
        for b, (idx_b, in_b, out_b, sg_b, sw_b) in enumerate(bufs):
            idx_n, in_n, _, sg_n, _ = bufs[1 - b]
            c = cc * 2 + b
            xoff = xbase + c * XR_CHUNK
            off = base + c * CHUNK

            @pl.when(c + 1 < N_CHUNKS)
            def _():
                pltpu.sync_copy(x_hbm.at[pl.ds(off + CHUNK, CHUNK)], idx_n)
                pltpu.make_async_copy(table_hbm.at[idx_n], in_n, sg_n).start()

            pltpu.make_async_copy(table_hbm.at[idx_b], in_b, sg_b).wait()

            @pl.when(c >= 2)
            def _():
                pltpu.make_async_copy(
                    out_b, out_hbm.at[pl.ds(xoff, XR_CHUNK)], sw_b
                ).wait()

            compute(in_b, out_b)
            pltpu.make_async_copy(
                out_b, out_hbm.at[pl.ds(xoff, XR_CHUNK)], sw_b
            ).start()
        return carry

    lax.fori_loop(0, N_CHUNKS // 2, pair_body, 0)

    for b, (_, _, out_b, _, sw_b) in enumerate(bufs):
        xoff = xbase + (N_CHUNKS - 2 + b) * XR_CHUNK
        pltpu.make_async_copy(
            out_b, out_hbm.at[pl.ds(xoff, XR_CHUNK)], sw_b
        ).wait()


@jax.jit
def kernel(x, table, bias):
    mesh = plsc.VectorSubcoreMesh(core_axis_name="c", subcore_axis_name="s")
    run = pl.kernel(
        _body,
        out_type=jax.ShapeDtypeStruct((B_ROWS, N_FEAT, EMB), jnp.float32),
        mesh=mesh,
        compiler_params=pltpu.CompilerParams(
            needs_layout_passes=False, use_tc_tiling_on_sc=False
        ),
        scratch_types=[
            pltpu.VMEM((CHUNK,), jnp.int32),
            pltpu.VMEM((CHUNK,), jnp.int32),
            pltpu.VMEM((CHUNK, EMB), jnp.float32),
            pltpu.VMEM((CHUNK, EMB), jnp.float32),
            pltpu.VMEM((XR_CHUNK, N_FEAT, EMB), jnp.float32),
            pltpu.VMEM((XR_CHUNK, N_FEAT, EMB), jnp.float32),
            pltpu.VMEM((EMB,), jnp.float32),
            pltpu.VMEM((EMB * L,), jnp.float32),
            pltpu.SemaphoreType.DMA,
            pltpu.SemaphoreType.DMA,
            pltpu.SemaphoreType.DMA,
            pltpu.SemaphoreType.DMA,
        ],
    )
    xf = x.reshape(-1).astype(jnp.int32)
    return run(xf, table, bias)
